# baseline (speedup 1.0000x reference)
import functools

import jax
import jax.numpy as jnp
from jax import lax
from jax.experimental import pallas as pl
from jax.experimental.pallas import tpu as pltpu
from jax.experimental.pallas import tpu_sc as plsc

ND = 2
NOISE = 0.5

_K = 128


def _make_global_indices(B, H, W):
    ii, jj = jnp.meshgrid(jnp.arange(H, dtype=jnp.float32),
                          jnp.arange(W, dtype=jnp.float32), indexing='ij')
    k1, k2 = jax.random.split(jax.random.key(123))
    n_i = jax.random.normal(k1, (B, H, W, 1), dtype=jnp.float32)
    n_j = jax.random.normal(k2, (B, H, W, 1), dtype=jnp.float32)
    A_i = ii[None, :, :, None] + n_i * NOISE
    A_j = jj[None, :, :, None] + n_j * NOISE
    A_i = jnp.floor(A_i + 0.4999).astype(jnp.int32)[..., 0]
    A_j = jnp.floor(A_j + 0.4999).astype(jnp.int32)[..., 0]
    A_i = jnp.clip(A_i, 0, H - 1)
    A_j = jnp.clip(A_j, 0, W - 1)
    flat = A_i * W + A_j
    g = flat.reshape(B, H * W) + (jnp.arange(B, dtype=jnp.int32) * (H * W))[:, None]
    return g.reshape(-1)


@functools.partial(jax.jit, static_argnums=(2, 3, 4))
def _sc_gather(img2, gidx, R, C, NW):
    rows_per_w = R // NW
    ch = rows_per_w // _K
    nc = NW // 16
    mesh = plsc.VectorSubcoreMesh(core_axis_name="c", subcore_axis_name="s")

    @functools.partial(
        pl.kernel, mesh=mesh,
        compiler_params=pltpu.CompilerParams(use_tc_tiling_on_sc=False),
        out_type=jax.ShapeDtypeStruct((R, C), jnp.float32),
        scratch_types=[
            pltpu.VMEM((ch, _K), jnp.int32),
            pltpu.VMEM((_K, C), jnp.float32),
            pltpu.SemaphoreType.DMA,
        ],
    )
    def k(img_hbm, idx_hbm, out_hbm, idx_v, buf, sem):
        wid = lax.axis_index("s") * nc + lax.axis_index("c")
        base = wid * rows_per_w
        pltpu.sync_copy(idx_hbm.at[wid], idx_v)

        def body(g, carry):
            pltpu.async_copy(img_hbm.at[idx_v.at[g]], buf, sem).wait()
            pltpu.sync_copy(buf, out_hbm.at[pl.ds(base + g * _K, _K)])
            return carry

        lax.fori_loop(0, ch, body, 0)

    return k(img2, gidx.reshape(NW, ch, _K))


def kernel(image):
    B, H, W, C = image.shape
    R = B * H * W
    NW = 32
    gidx = _make_global_indices(B, H, W)
    out = _sc_gather(image.reshape(R, C), gidx, R, C, NW)
    return out.reshape(B, H, W, C)


# speedup vs baseline: 3.1468x; 3.1468x over previous
import functools

import jax
import jax.numpy as jnp
from jax import lax
from jax.experimental import pallas as pl
from jax.experimental.pallas import tpu as pltpu
from jax.experimental.pallas import tpu_sc as plsc

ND = 2
NOISE = 0.5

HALO = 4
TI = 16
CSLAB = 8
LANES = 16


def _make_index_planes(B, H, W):
    ii, jj = jnp.meshgrid(jnp.arange(H, dtype=jnp.float32),
                          jnp.arange(W, dtype=jnp.float32), indexing='ij')
    k1, k2 = jax.random.split(jax.random.key(123))
    n_i = jax.random.normal(k1, (B, H, W, 1), dtype=jnp.float32)
    n_j = jax.random.normal(k2, (B, H, W, 1), dtype=jnp.float32)
    A_i = ii[None, :, :, None] + n_i * NOISE
    A_j = jj[None, :, :, None] + n_j * NOISE
    A_i = jnp.floor(A_i + 0.4999).astype(jnp.int32)[..., 0]
    A_j = jnp.floor(A_j + 0.4999).astype(jnp.int32)[..., 0]
    A_i = jnp.clip(A_i, 0, H - 1)
    A_j = jnp.clip(A_j, 0, W - 1)
    return A_i, A_j


@functools.partial(jax.jit, static_argnums=(3,))
def _sc_scramble(img_t, ai, aj, dims):
    B, H, C, W = dims
    nblocks = H // TI
    nslabs = C // CSLAB
    cgroups = 32 // B
    slabs_per_w = nslabs // cgroups
    njv = W // LANES
    mesh = plsc.VectorSubcoreMesh(core_axis_name="c", subcore_axis_name="s")

    @functools.partial(
        pl.kernel, mesh=mesh,
        compiler_params=pltpu.CompilerParams(needs_layout_passes=False),
        out_type=jax.ShapeDtypeStruct((B, H, C, W), jnp.float32),
        scratch_types=[
            pltpu.VMEM((TI + 2 * HALO, CSLAB, W), jnp.float32),
            pltpu.VMEM((TI, CSLAB, W), jnp.float32),
            pltpu.VMEM((TI, W), jnp.int32),
            pltpu.VMEM((TI, W), jnp.int32),
        ],
    )
    def k(img_hbm, ai_hbm, aj_hbm, out_hbm, slab, outb, qa, qj):
        wid = lax.axis_index("s") * 2 + lax.axis_index("c")
        b = wid // cgroups
        cg = wid % cgroups

        def block_body(blk, carry):
            i0 = blk * TI
            s = jnp.clip(i0 - HALO, 0, H - (TI + 2 * HALO))
            pltpu.sync_copy(ai_hbm.at[b, pl.ds(i0, TI)], qa)
            pltpu.sync_copy(aj_hbm.at[b, pl.ds(i0, TI)], qj)
            for k_ in range(slabs_per_w):
                c0 = (cg * slabs_per_w + k_) * CSLAB
                pltpu.sync_copy(
                    img_hbm.at[b, pl.ds(s, TI + 2 * HALO), pl.ds(c0, CSLAB)],
                    slab)

                def row_body(i_loc, rcarry):
                    for jv in range(njv):
                        rows = qa[i_loc, pl.ds(jv * LANES, LANES)] - s
                        cols = qj[i_loc, pl.ds(jv * LANES, LANES)]
                        for c in range(CSLAB):
                            cvec = jnp.full((LANES,), c, jnp.int32)
                            v = plsc.load_gather(slab, [rows, cvec, cols])
                            outb[i_loc, c, pl.ds(jv * LANES, LANES)] = v
                    return rcarry

                lax.fori_loop(0, TI, row_body, 0)
                pltpu.sync_copy(
                    outb, out_hbm.at[b, pl.ds(i0, TI), pl.ds(c0, CSLAB)])
            return carry

        lax.fori_loop(0, nblocks, block_body, 0)

    return k(img_t, ai, aj)


def kernel(image):
    B, H, W, C = image.shape
    ai, aj = _make_index_planes(B, H, W)
    img_t = jnp.transpose(image, (0, 1, 3, 2))
    out_t = _sc_scramble(img_t, ai, aj, (B, H, C, W))
    return jnp.transpose(out_t, (0, 1, 3, 2))


# speedup vs baseline: 4.5682x; 1.4517x over previous
import functools

import jax
import jax.numpy as jnp
from jax import lax
from jax.experimental import pallas as pl
from jax.experimental.pallas import tpu as pltpu
from jax.experimental.pallas import tpu_sc as plsc

ND = 2
NOISE = 0.5

RING = 32
TI = 8
CSLAB = 8
LANES = 16


def _make_packed_indices(B, H, W):
    ii, jj = jnp.meshgrid(jnp.arange(H, dtype=jnp.float32),
                          jnp.arange(W, dtype=jnp.float32), indexing='ij')
    k1, k2 = jax.random.split(jax.random.key(123))
    n_i = jax.random.normal(k1, (B, H, W, 1), dtype=jnp.float32)
    n_j = jax.random.normal(k2, (B, H, W, 1), dtype=jnp.float32)
    A_i = ii[None, :, :, None] + n_i * NOISE
    A_j = jj[None, :, :, None] + n_j * NOISE
    A_i = jnp.floor(A_i + 0.4999).astype(jnp.int32)[..., 0]
    A_j = jnp.floor(A_j + 0.4999).astype(jnp.int32)[..., 0]
    A_i = jnp.clip(A_i, 0, H - 1)
    A_j = jnp.clip(A_j, 0, W - 1)
    return (A_i % RING) * 256 + A_j


@functools.partial(jax.jit, static_argnums=(2,))
def _sc_scramble(img_t, q, dims):
    B, H, C, W = dims
    nblk = H // TI
    cgroups = 32 // B
    slabs_per_w = (C // CSLAB) // cgroups
    njv = W // LANES
    mesh = plsc.VectorSubcoreMesh(core_axis_name="c", subcore_axis_name="s")

    @functools.partial(
        pl.kernel, mesh=mesh,
        compiler_params=pltpu.CompilerParams(needs_layout_passes=False),
        out_type=jax.ShapeDtypeStruct((B, H, C, W), jnp.float32),
        scratch_types=[
            pltpu.VMEM((RING, CSLAB, W), jnp.float32),
            pltpu.VMEM((TI, CSLAB, W), jnp.float32),
            pltpu.VMEM((TI, CSLAB, W), jnp.float32),
            pltpu.VMEM((TI, W), jnp.int32),
            pltpu.VMEM((TI, W), jnp.int32),
            pltpu.SemaphoreType.DMA,
            pltpu.SemaphoreType.DMA,
            pltpu.SemaphoreType.DMA,
            pltpu.SemaphoreType.DMA,
        ],
    )
    def k(img_hbm, q_hbm, out_hbm, slab, ob0, ob1, qb0, qb1,
          in_sem, q_sem, o_sem0, o_sem1):
        wid = lax.axis_index("s") * 2 + lax.axis_index("c")
        b = wid // cgroups
        cg = wid % cgroups

        def in_wait():
            pltpu.make_async_copy(
                img_hbm.at[b, pl.ds(0, TI), pl.ds(0, CSLAB)],
                slab.at[pl.ds(0, TI)], in_sem).wait()

        def q_wait():
            pltpu.make_async_copy(q_hbm.at[b, pl.ds(0, TI)], qb0, q_sem).wait()

        def strip_body(k_, carry):
            c0 = cg * (slabs_per_w * CSLAB) + k_ * CSLAB
            pltpu.async_copy(img_hbm.at[b, pl.ds(0, TI), pl.ds(c0, CSLAB)],
                             slab.at[pl.ds(0, TI)], in_sem)
            pltpu.async_copy(img_hbm.at[b, pl.ds(TI, TI), pl.ds(c0, CSLAB)],
                             slab.at[pl.ds(TI, TI)], in_sem)
            pltpu.async_copy(q_hbm.at[b, pl.ds(0, TI)], qb0, q_sem)

            def half(blk, ob, qb, qb_next, o_sem):
                i0 = blk * TI
                @pl.when(blk < nblk - 2)
                def _():
                    r0 = i0 + 2 * TI
                    slot = lax.rem(r0, RING)
                    pltpu.async_copy(
                        img_hbm.at[b, pl.ds(r0, TI), pl.ds(c0, CSLAB)],
                        slab.at[pl.ds(slot, TI)], in_sem)

                @pl.when(blk < nblk - 1)
                def _():
                    pltpu.async_copy(q_hbm.at[b, pl.ds(i0 + TI, TI)],
                                     qb_next, q_sem)

                @pl.when(blk == 0)
                def _():
                    in_wait()
                @pl.when(blk < nblk - 1)
                def _():
                    in_wait()
                q_wait()

                @pl.when(blk >= 2)
                def _():
                    pltpu.make_async_copy(
                        ob, out_hbm.at[b, pl.ds(0, TI), pl.ds(0, CSLAB)],
                        o_sem).wait()

                def row_body(i_loc, rcarry):
                    for jv in range(njv):
                        qv = qb[i_loc, pl.ds(jv * LANES, LANES)]
                        rows = lax.shift_right_logical(qv, 8)
                        cols = lax.bitwise_and(qv, 255)
                        for c in range(CSLAB):
                            cvec = jnp.full((LANES,), c, jnp.int32)
                            v = plsc.load_gather(slab, [rows, cvec, cols])
                            ob[i_loc, c, pl.ds(jv * LANES, LANES)] = v
                    return rcarry

                lax.fori_loop(0, TI, row_body, 0)
                pltpu.async_copy(
                    ob, out_hbm.at[b, pl.ds(i0, TI), pl.ds(c0, CSLAB)], o_sem)

            def blk2_body(u, c2):
                half(2 * u, ob0, qb0, qb1, o_sem0)
                half(2 * u + 1, ob1, qb1, qb0, o_sem1)
                return c2

            lax.fori_loop(0, nblk // 2, blk2_body, 0)
            pltpu.make_async_copy(
                ob0, out_hbm.at[b, pl.ds(0, TI), pl.ds(0, CSLAB)],
                o_sem0).wait()
            pltpu.make_async_copy(
                ob1, out_hbm.at[b, pl.ds(0, TI), pl.ds(0, CSLAB)],
                o_sem1).wait()
            return carry

        lax.fori_loop(0, slabs_per_w, strip_body, 0)

    return k(img_t, q)


def kernel(image):
    B, H, W, C = image.shape
    q = _make_packed_indices(B, H, W)
    img_t = jnp.transpose(image, (0, 1, 3, 2))
    out_t = _sc_scramble(img_t, q, (B, H, C, W))
    return jnp.transpose(out_t, (0, 1, 3, 2))


# speedup vs baseline: 8.1556x; 1.7853x over previous
import functools

import jax
import jax.numpy as jnp
from jax import lax
from jax.experimental import pallas as pl
from jax.experimental.pallas import tpu as pltpu
from jax.experimental.pallas import tpu_sc as plsc

ND = 2
NOISE = 0.5

RING = 32
TI = 8
CSLAB = 8
LANES = 16


def _make_packed_indices(B, H, W):
    ii, jj = jnp.meshgrid(jnp.arange(H, dtype=jnp.float32),
                          jnp.arange(W, dtype=jnp.float32), indexing='ij')
    k1, k2 = jax.random.split(jax.random.key(123))
    n_i = jax.random.normal(k1, (B, H, W, 1), dtype=jnp.float32)
    n_j = jax.random.normal(k2, (B, H, W, 1), dtype=jnp.float32)
    A_i = ii[None, :, :, None] + n_i * NOISE
    A_j = jj[None, :, :, None] + n_j * NOISE
    A_i = jnp.floor(A_i + 0.4999).astype(jnp.int32)[..., 0]
    A_j = jnp.floor(A_j + 0.4999).astype(jnp.int32)[..., 0]
    A_i = jnp.clip(A_i, 0, H - 1)
    A_j = jnp.clip(A_j, 0, W - 1)
    return (A_i % RING) * 256 + A_j


@functools.partial(jax.jit, static_argnums=(2,))
def _sc_scramble(img_t, q, dims):
    B, H, C, W = dims
    nblk = H // TI
    cgroups = 32 // B
    slabs_per_w = (C // CSLAB) // cgroups
    njv = W // LANES
    mesh = plsc.VectorSubcoreMesh(core_axis_name="c", subcore_axis_name="s")

    @functools.partial(
        pl.kernel, mesh=mesh,
        compiler_params=pltpu.CompilerParams(needs_layout_passes=False),
        out_type=jax.ShapeDtypeStruct((B, H, C, W), jnp.float32),
        scratch_types=[
            pltpu.VMEM((RING, CSLAB, W), jnp.float32),
            pltpu.VMEM((TI, CSLAB, W), jnp.float32),
            pltpu.VMEM((TI, CSLAB, W), jnp.float32),
            pltpu.VMEM((TI, W), jnp.int32),
            pltpu.VMEM((TI, W), jnp.int32),
            pltpu.SemaphoreType.DMA,
            pltpu.SemaphoreType.DMA,
            pltpu.SemaphoreType.DMA,
            pltpu.SemaphoreType.DMA,
        ],
    )
    def k(img_hbm, q_hbm, out_hbm, slab, ob0, ob1, qb0, qb1,
          in_sem, q_sem, o_sem0, o_sem1):
        wid = lax.axis_index("s") * 2 + lax.axis_index("c")
        b = wid // cgroups
        cg = wid % cgroups

        def in_wait():
            pltpu.make_async_copy(
                img_hbm.at[b, pl.ds(0, TI), pl.ds(0, CSLAB)],
                slab.at[pl.ds(0, TI)], in_sem).wait()

        def q_wait():
            pltpu.make_async_copy(q_hbm.at[b, pl.ds(0, TI)], qb0, q_sem).wait()

        def strip_body(k_, carry):
            c0 = cg * (slabs_per_w * CSLAB) + k_ * CSLAB
            pltpu.async_copy(img_hbm.at[b, pl.ds(0, TI), pl.ds(c0, CSLAB)],
                             slab.at[pl.ds(0, TI)], in_sem)
            pltpu.async_copy(img_hbm.at[b, pl.ds(TI, TI), pl.ds(c0, CSLAB)],
                             slab.at[pl.ds(TI, TI)], in_sem)
            pltpu.async_copy(q_hbm.at[b, pl.ds(0, TI)], qb0, q_sem)

            def half(blk, ob, qb, qb_next, o_sem):
                i0 = blk * TI
                @pl.when(blk < nblk - 2)
                def _():
                    r0 = i0 + 2 * TI
                    slot = lax.rem(r0, RING)
                    pltpu.async_copy(
                        img_hbm.at[b, pl.ds(r0, TI), pl.ds(c0, CSLAB)],
                        slab.at[pl.ds(slot, TI)], in_sem)

                @pl.when(blk < nblk - 1)
                def _():
                    pltpu.async_copy(q_hbm.at[b, pl.ds(i0 + TI, TI)],
                                     qb_next, q_sem)

                @pl.when(blk == 0)
                def _():
                    in_wait()
                @pl.when(blk < nblk - 1)
                def _():
                    in_wait()
                q_wait()

                @pl.when(blk >= 2)
                def _():
                    pltpu.make_async_copy(
                        ob, out_hbm.at[b, pl.ds(0, TI), pl.ds(0, CSLAB)],
                        o_sem).wait()

                @plsc.parallel_loop(0, TI, unroll=2)
                def row_body(i_loc):
                    for jv in range(njv):
                        qv = qb[i_loc, pl.ds(jv * LANES, LANES)]
                        rows = lax.shift_right_logical(qv, 8)
                        cols = lax.bitwise_and(qv, 255)
                        for c in range(CSLAB):
                            cvec = jnp.full((LANES,), c, jnp.int32)
                            v = plsc.load_gather(slab, [rows, cvec, cols])
                            ob[i_loc, c, pl.ds(jv * LANES, LANES)] = v
                pltpu.async_copy(
                    ob, out_hbm.at[b, pl.ds(i0, TI), pl.ds(c0, CSLAB)], o_sem)

            def blk2_body(u, c2):
                half(2 * u, ob0, qb0, qb1, o_sem0)
                half(2 * u + 1, ob1, qb1, qb0, o_sem1)
                return c2

            lax.fori_loop(0, nblk // 2, blk2_body, 0)
            pltpu.make_async_copy(
                ob0, out_hbm.at[b, pl.ds(0, TI), pl.ds(0, CSLAB)],
                o_sem0).wait()
            pltpu.make_async_copy(
                ob1, out_hbm.at[b, pl.ds(0, TI), pl.ds(0, CSLAB)],
                o_sem1).wait()
            return carry

        lax.fori_loop(0, slabs_per_w, strip_body, 0)

    return k(img_t, q)


def kernel(image):
    B, H, W, C = image.shape
    q = _make_packed_indices(B, H, W)
    img_t = jnp.transpose(image, (0, 1, 3, 2))
    out_t = _sc_scramble(img_t, q, (B, H, C, W))
    return jnp.transpose(out_t, (0, 1, 3, 2))
